# baseline (speedup 1.0000x reference)
import functools

import jax
import jax.numpy as jnp
from jax import lax
from jax.experimental import pallas as pl
from jax.experimental.pallas import tpu as pltpu
from jax.experimental.pallas import tpu_sc as plsc

B = 16384
D = 64
N = 1000000
HALF = 500096
PR = HALF
NC = 2
NS = 16
NW = NC * NS
TPW = B // NW
C = 64
NCH = TPW // C
NG = C // 16
NBLK = 3907


def _convert_tables(node_t, edge_t):
    blk_in = lambda half: pl.BlockSpec((D, 128), lambda i: (0, i + half * NBLK))
    out_spec = pl.BlockSpec((128, 128), lambda i: (i, 0))

    def body(na, nb, ea, eb, n_out, e_out):
        n_out[...] = jnp.concatenate([na[...].T, nb[...].T], axis=1)
        e_out[...] = jnp.concatenate([ea[...].T, eb[...].T], axis=1)

    conv = pl.pallas_call(
        body,
        grid=(NBLK,),
        in_specs=[blk_in(0), blk_in(1), blk_in(0), blk_in(1)],
        out_specs=[out_spec, out_spec],
        out_shape=[
            jax.ShapeDtypeStruct((PR, 128), jnp.float32),
            jax.ShapeDtypeStruct((PR, 128), jnp.float32),
        ],
        compiler_params=pltpu.CompilerParams(
            dimension_semantics=("arbitrary",)),
    )
    return conv(node_t, node_t, edge_t, edge_t)


def _make_sc_call():
    mesh = plsc.VectorSubcoreMesh(core_axis_name="c", subcore_axis_name="s")

    idx_buf = lambda: pltpu.VMEM((NCH, C), jnp.int32)
    row_buf = lambda: pltpu.VMEM((C, 128), jnp.float32)

    @functools.partial(
        pl.kernel,
        mesh=mesh,
        out_type=jax.ShapeDtypeStruct((B,), jnp.float32),
        compiler_params=pltpu.CompilerParams(
            needs_layout_passes=False, use_tc_tiling_on_sc=True),
        scratch_types=[
            [idx_buf() for _ in range(6)],
            [idx_buf() for _ in range(6)],
            [[row_buf() for _ in range(6)] for _ in range(2)],
            pltpu.VMEM((TPW,), jnp.float32),
            pltpu.SemaphoreType.DMA,
            pltpu.SemaphoreType.DMA,
        ],
    )
    def sc_kernel(hp_i, rp_i, tp_i, hn_i, rn_i, tn_i, node2, edge2, out,
                  raw, adj, bufs, out_v, sem0, sem1):
        wid = lax.axis_index("s") * NC + lax.axis_index("c")
        sems = [sem0, sem1]
        tables = [node2, edge2, node2, node2, edge2, node2]

        for hbm_i, vmem_i in zip([hp_i, rp_i, tp_i, hn_i, rn_i, tn_i], raw):
            pltpu.sync_copy(hbm_i.at[pl.ds(wid * NCH, NCH)], vmem_i)

        for o in range(6):
            for c in range(NCH):
                for k in range(C // 16):
                    sl = pl.ds(k * 16, 16)
                    v = raw[o][c, sl]
                    adj[o][c, sl] = jnp.where(v >= HALF, v - HALF, v)

        def fire(c, slot):
            return [pltpu.async_copy(tables[o].at[adj[o].at[c]],
                                     bufs[slot][o], sems[slot])
                    for o in range(6)]

        def compute_chunk(c, slot):
            bs = bufs[slot]

            def side_dist(hb, rb, tb, ho, ro, to, g):
                rows = lax.iota(jnp.int32, 16) + g * 16
                cols = []
                for o in (ho, ro, to):
                    v = raw[o][c, pl.ds(g * 16, 16)]
                    cols.append(jnp.where(v >= HALF, 64, 0))
                acc = jnp.zeros((16,), jnp.float32)
                for d in range(D):
                    hv = plsc.load_gather(hb, [rows, cols[0] + d])
                    rv = plsc.load_gather(rb, [rows, cols[1] + d])
                    tv = plsc.load_gather(tb, [rows, cols[2] + d])
                    acc = acc + jnp.abs(hv + rv - tv)
                return acc

            def group_body(g, _):
                pos_d = side_dist(bs[0], bs[1], bs[2], 0, 1, 2, g)
                neg_d = side_dist(bs[3], bs[4], bs[5], 3, 4, 5, g)
                loss = jnp.maximum(pos_d - neg_d + 1.0, 0.0)
                out_v[pl.ds(c * C + g * 16, 16)] = loss
                return 0

            lax.fori_loop(0, NG, group_body, 0)

        pending = {0: fire(0, 0), 1: fire(1, 1)}
        for c in range(NCH):
            slot = c % 2
            for cp in pending[slot]:
                cp.wait()
            compute_chunk(c, slot)
            if c + 2 < NCH:
                pending[slot] = fire(c + 2, slot)

        pltpu.sync_copy(out_v, out.at[pl.ds(wid * TPW, TPW)])

    return sc_kernel


def kernel(pos_triplets, neg_triplets, node_em, edge_em):
    pos = pos_triplets.astype(jnp.int32)
    neg = neg_triplets.astype(jnp.int32)
    node2, edge2 = _convert_tables(node_em.T, edge_em.T)
    idx_arrays = [
        pos[:, 0].reshape(B // C, C),
        pos[:, 1].reshape(B // C, C),
        pos[:, 2].reshape(B // C, C),
        neg[:, 0].reshape(B // C, C),
        neg[:, 1].reshape(B // C, C),
        neg[:, 2].reshape(B // C, C),
    ]
    sc = _make_sc_call()
    return sc(*idx_arrays, node2, edge2)


# speedup vs baseline: 1.9617x; 1.9617x over previous
import functools

import jax
import jax.numpy as jnp
from jax import lax
from jax.experimental import pallas as pl
from jax.experimental.pallas import tpu as pltpu
from jax.experimental.pallas import tpu_sc as plsc

B = 16384
D = 64
N = 1000000
HALF = 500096
PR = HALF
NC = 2
NS = 16
NW = NC * NS
TPW = B // NW
C = 64
NCH = TPW // C
NG = C // 16
NBLK = 3907


def _pair_table(t):
    lo = t[:HALF]
    hi = jnp.pad(t[HALF:], ((0, 2 * HALF - N), (0, 0)))
    return jnp.concatenate([lo, hi], axis=1)


def _make_sc_call():
    mesh = plsc.VectorSubcoreMesh(core_axis_name="c", subcore_axis_name="s")

    idx_buf = lambda: pltpu.VMEM((NCH, C), jnp.int32)
    row_buf = lambda: pltpu.VMEM((C, 128), jnp.float32)

    @functools.partial(
        pl.kernel,
        mesh=mesh,
        out_type=jax.ShapeDtypeStruct((B,), jnp.float32),
        compiler_params=pltpu.CompilerParams(
            needs_layout_passes=False, use_tc_tiling_on_sc=True),
        scratch_types=[
            [idx_buf() for _ in range(6)],
            [idx_buf() for _ in range(6)],
            [[row_buf() for _ in range(6)] for _ in range(2)],
            pltpu.VMEM((TPW,), jnp.float32),
            pltpu.SemaphoreType.DMA,
            pltpu.SemaphoreType.DMA,
        ],
    )
    def sc_kernel(hp_i, rp_i, tp_i, hn_i, rn_i, tn_i, node2, edge2, out,
                  raw, adj, bufs, out_v, sem0, sem1):
        wid = lax.axis_index("s") * NC + lax.axis_index("c")
        sems = [sem0, sem1]
        tables = [node2, edge2, node2, node2, edge2, node2]

        for hbm_i, vmem_i in zip([hp_i, rp_i, tp_i, hn_i, rn_i, tn_i], raw):
            pltpu.sync_copy(hbm_i.at[pl.ds(wid * NCH, NCH)], vmem_i)

        for o in range(6):
            for c in range(NCH):
                for k in range(C // 16):
                    sl = pl.ds(k * 16, 16)
                    v = raw[o][c, sl]
                    adj[o][c, sl] = jnp.where(v >= HALF, v - HALF, v)

        def fire(c, slot):
            return [pltpu.async_copy(tables[o].at[adj[o].at[c]],
                                     bufs[slot][o], sems[slot])
                    for o in range(6)]

        def compute_chunk(c, slot):
            bs = bufs[slot]

            def side_dist(hb, rb, tb, ho, ro, to, g):
                rows = lax.iota(jnp.int32, 16) + g * 16
                cols = []
                for o in (ho, ro, to):
                    v = raw[o][c, pl.ds(g * 16, 16)]
                    cols.append(jnp.where(v >= HALF, 64, 0))
                acc = jnp.zeros((16,), jnp.float32)
                for d in range(D):
                    hv = plsc.load_gather(hb, [rows, cols[0] + d])
                    rv = plsc.load_gather(rb, [rows, cols[1] + d])
                    tv = plsc.load_gather(tb, [rows, cols[2] + d])
                    acc = acc + jnp.abs(hv + rv - tv)
                return acc

            def group_body(g, _):
                pos_d = side_dist(bs[0], bs[1], bs[2], 0, 1, 2, g)
                neg_d = side_dist(bs[3], bs[4], bs[5], 3, 4, 5, g)
                loss = jnp.maximum(pos_d - neg_d + 1.0, 0.0)
                out_v[pl.ds(c * C + g * 16, 16)] = loss
                return 0

            lax.fori_loop(0, NG, group_body, 0)

        pending = {0: fire(0, 0), 1: fire(1, 1)}
        for c in range(NCH):
            slot = c % 2
            for cp in pending[slot]:
                cp.wait()
            compute_chunk(c, slot)
            if c + 2 < NCH:
                pending[slot] = fire(c + 2, slot)

        pltpu.sync_copy(out_v, out.at[pl.ds(wid * TPW, TPW)])

    return sc_kernel


def kernel(pos_triplets, neg_triplets, node_em, edge_em):
    pos = pos_triplets.astype(jnp.int32)
    neg = neg_triplets.astype(jnp.int32)
    node2 = _pair_table(node_em)
    edge2 = _pair_table(edge_em)
    idx_arrays = [
        pos[:, 0].reshape(B // C, C),
        pos[:, 1].reshape(B // C, C),
        pos[:, 2].reshape(B // C, C),
        neg[:, 0].reshape(B // C, C),
        neg[:, 1].reshape(B // C, C),
        neg[:, 2].reshape(B // C, C),
    ]
    sc = _make_sc_call()
    return sc(*idx_arrays, node2, edge2)


# speedup vs baseline: 2.3373x; 1.1914x over previous
import functools

import jax
import jax.numpy as jnp
from jax import lax
from jax.experimental import pallas as pl
from jax.experimental.pallas import tpu as pltpu
from jax.experimental.pallas import tpu_sc as plsc

B = 16384
D = 64
N = 1000000
NC = 2
NS = 16
NW = NC * NS
TPW = B // NW
C = 64
NCH = TPW // C
NG = C // 16


def _make_sc_call():
    mesh = plsc.VectorSubcoreMesh(core_axis_name="c", subcore_axis_name="s")

    idx_buf = lambda: pltpu.VMEM((NCH, C), jnp.int32)
    row_buf = lambda: pltpu.VMEM((C, 128), jnp.float32)

    @functools.partial(
        pl.kernel,
        mesh=mesh,
        out_type=jax.ShapeDtypeStruct((B,), jnp.float32),
        compiler_params=pltpu.CompilerParams(
            needs_layout_passes=False, use_tc_tiling_on_sc=True),
        scratch_types=[
            [idx_buf() for _ in range(6)],
            [[row_buf() for _ in range(6)] for _ in range(2)],
            pltpu.VMEM((256,), jnp.float32),
            pltpu.VMEM((TPW,), jnp.float32),
            pltpu.SemaphoreType.DMA,
            pltpu.SemaphoreType.DMA,
        ],
    )
    def sc_kernel(hp_i, rp_i, tp_i, hn_i, rn_i, tn_i, node2, edge2, out,
                  raw, bufs, tr_v, out_v, sem0, sem1):
        wid = lax.axis_index("s") * NC + lax.axis_index("c")
        iota16 = lax.iota(jnp.int32, 16)
        sems = [sem0, sem1]
        tables = [node2, edge2, node2, node2, edge2, node2]

        for hbm_i, vmem_i in zip([hp_i, rp_i, tp_i, hn_i, rn_i, tn_i], raw):
            pltpu.sync_copy(hbm_i.at[pl.ds(wid * NCH, NCH)], vmem_i)

        def fire(c, slot):
            return [pltpu.async_copy(tables[o].at[raw[o].at[c]],
                                     bufs[slot][o], sems[slot])
                    for o in range(6)]

        def compute_chunk(c, slot):
            bs = bufs[slot]

            def side_dist(hv, rv, tv, base):
                for i in range(16):
                    r_ = base + i
                    parts = []
                    for k in range(D // 16):
                        sl = pl.ds(k * 16, 16)
                        parts.append(
                            jnp.abs(hv[r_, sl] + rv[r_, sl] - tv[r_, sl]))
                    part = (parts[0] + parts[1]) + (parts[2] + parts[3])
                    plsc.store_scatter(tr_v, [iota16 * 16 + i], part)
                rows = [tr_v[pl.ds(j * 16, 16)] for j in range(16)]
                while len(rows) > 1:
                    rows = [rows[2 * j] + rows[2 * j + 1]
                            for j in range(len(rows) // 2)]
                return rows[0]

            def group_body(g, _):
                base = g * 16
                pos_d = side_dist(bs[0], bs[1], bs[2], base)
                neg_d = side_dist(bs[3], bs[4], bs[5], base)
                loss = jnp.maximum(pos_d - neg_d + 1.0, 0.0)
                out_v[pl.ds(c * C + g * 16, 16)] = loss
                return 0

            lax.fori_loop(0, NG, group_body, 0)

        pending = {0: fire(0, 0), 1: fire(1, 1)}
        for c in range(NCH):
            slot = c % 2
            for cp in pending[slot]:
                cp.wait()
            compute_chunk(c, slot)
            if c + 2 < NCH:
                pending[slot] = fire(c + 2, slot)

        pltpu.sync_copy(out_v, out.at[pl.ds(wid * TPW, TPW)])

    return sc_kernel


def kernel(pos_triplets, neg_triplets, node_em, edge_em):
    pos = pos_triplets.astype(jnp.int32)
    neg = neg_triplets.astype(jnp.int32)
    node2 = jnp.pad(node_em, ((0, 0), (0, 64)))
    edge2 = jnp.pad(edge_em, ((0, 0), (0, 64)))
    idx_arrays = [
        pos[:, 0].reshape(B // C, C),
        pos[:, 1].reshape(B // C, C),
        pos[:, 2].reshape(B // C, C),
        neg[:, 0].reshape(B // C, C),
        neg[:, 1].reshape(B // C, C),
        neg[:, 2].reshape(B // C, C),
    ]
    sc = _make_sc_call()
    return sc(*idx_arrays, node2, edge2)


# speedup vs baseline: 2.6547x; 1.1358x over previous
import functools

import jax
import jax.numpy as jnp
from jax import lax
from jax.experimental import pallas as pl
from jax.experimental.pallas import tpu as pltpu
from jax.experimental.pallas import tpu_sc as plsc

B = 16384
D = 64
N = 1000000
NC = 2
NS = 16
NW = NC * NS
TPW = B // NW
C = 64
NCH = TPW // C
NG = C // 16


def _make_sc_call():
    mesh = plsc.VectorSubcoreMesh(core_axis_name="c", subcore_axis_name="s")

    idx_buf = lambda: pltpu.VMEM((NCH, C), jnp.int32)
    row_buf = lambda: pltpu.VMEM((C, 128), jnp.float32)

    @functools.partial(
        pl.kernel,
        mesh=mesh,
        out_type=jax.ShapeDtypeStruct((B,), jnp.float32),
        compiler_params=pltpu.CompilerParams(
            needs_layout_passes=False, use_tc_tiling_on_sc=True),
        scratch_types=[
            [idx_buf() for _ in range(6)],
            [[row_buf() for _ in range(6)] for _ in range(2)],
            pltpu.VMEM((256,), jnp.float32),
            pltpu.VMEM((TPW,), jnp.float32),
            pltpu.SemaphoreType.DMA,
            pltpu.SemaphoreType.DMA,
        ],
    )
    def sc_kernel(hp_i, rp_i, tp_i, hn_i, rn_i, tn_i, comb, out,
                  raw, bufs, tr_v, out_v, sem0, sem1):
        wid = lax.axis_index("s") * NC + lax.axis_index("c")
        iota16 = lax.iota(jnp.int32, 16)
        sems = [sem0, sem1]

        for hbm_i, vmem_i in zip([hp_i, rp_i, tp_i, hn_i, rn_i, tn_i], raw):
            pltpu.sync_copy(hbm_i.at[pl.ds(wid * NCH, NCH)], vmem_i)

        def fire(c, slot):
            return [pltpu.async_copy(comb.at[raw[o].at[c]],
                                     bufs[slot][o], sems[slot])
                    for o in range(6)]

        def compute_chunk(c, slot):
            bs = bufs[slot]

            def side_dist(hv, rv, tv, base):
                for i in range(16):
                    r_ = base + i
                    parts = []
                    for k in range(D // 16):
                        sl = pl.ds(k * 16, 16)
                        sr = pl.ds(64 + k * 16, 16)
                        parts.append(
                            jnp.abs(hv[r_, sl] + rv[r_, sr] - tv[r_, sl]))
                    part = (parts[0] + parts[1]) + (parts[2] + parts[3])
                    plsc.store_scatter(tr_v, [iota16 * 16 + i], part)
                rows = [tr_v[pl.ds(j * 16, 16)] for j in range(16)]
                while len(rows) > 1:
                    rows = [rows[2 * j] + rows[2 * j + 1]
                            for j in range(len(rows) // 2)]
                return rows[0]

            def group_body(g, _):
                base = g * 16
                pos_d = side_dist(bs[0], bs[1], bs[2], base)
                neg_d = side_dist(bs[3], bs[4], bs[5], base)
                loss = jnp.maximum(pos_d - neg_d + 1.0, 0.0)
                out_v[pl.ds(c * C + g * 16, 16)] = loss
                return 0

            lax.fori_loop(0, NG, group_body, 0)

        pending = {0: fire(0, 0), 1: fire(1, 1)}
        for c in range(NCH):
            slot = c % 2
            for cp in pending[slot]:
                cp.wait()
            compute_chunk(c, slot)
            if c + 2 < NCH:
                pending[slot] = fire(c + 2, slot)

        pltpu.sync_copy(out_v, out.at[pl.ds(wid * TPW, TPW)])

    return sc_kernel


def kernel(pos_triplets, neg_triplets, node_em, edge_em):
    pos = pos_triplets.astype(jnp.int32)
    neg = neg_triplets.astype(jnp.int32)
    comb = jnp.concatenate([node_em, edge_em], axis=1)
    idx_arrays = [
        pos[:, 0].reshape(B // C, C),
        pos[:, 1].reshape(B // C, C),
        pos[:, 2].reshape(B // C, C),
        neg[:, 0].reshape(B // C, C),
        neg[:, 1].reshape(B // C, C),
        neg[:, 2].reshape(B // C, C),
    ]
    sc = _make_sc_call()
    return sc(*idx_arrays, comb)
